# Initial kernel scaffold; baseline (speedup 1.0000x reference)
import jax
import jax.numpy as jnp
from jax.experimental import pallas as pl


def kernel(hidden_states, gate_w, Wg, Wu, Wd):
    raise NotImplementedError("write your pallas kernel here")


# speedup vs baseline: 1.3824x; 1.3824x over previous
import functools

import jax
import jax.numpy as jnp
from jax.experimental import pallas as pl
from jax.experimental.pallas import tpu as pltpu

E = 8
TOPK = 2
H = 1024
F = 512
T = 2048
TT = 256
NT = T // TT


def _gate_kernel(x_ref, gw_ref, comb_ref):
    x = x_ref[...]
    gw = gw_ref[...]
    logits = jax.lax.dot_general(
        x, gw, (((1,), (1,)), ((), ())), preferred_element_type=jnp.float32)
    m = jnp.max(logits, axis=1, keepdims=True)
    p = jnp.exp(logits - m)
    p = p / jnp.sum(p, axis=1, keepdims=True)
    lane = jax.lax.broadcasted_iota(jnp.int32, p.shape, 1)
    m1 = jnp.max(p, axis=1, keepdims=True)
    i1 = jnp.min(jnp.where(p >= m1, lane, E), axis=1, keepdims=True)
    p2 = jnp.where(lane == i1, -jnp.inf, p)
    m2 = jnp.max(p2, axis=1, keepdims=True)
    i2 = jnp.min(jnp.where(p2 >= m2, lane, E), axis=1, keepdims=True)
    s = m1 + m2 + 1e-20
    comb = jnp.where(lane == i1, m1 / s, 0.0) + jnp.where(lane == i2, m2 / s, 0.0)
    comb_ref[...] = comb


def _moe_kernel(xb_ref, comb_ref, wg_ref, wu_ref, wd_ref, out_ref):
    e = pl.program_id(1)
    xb = xb_ref[...]
    wg = wg_ref[e]
    wu = wu_ref[e]
    wd = wd_ref[e]
    g = jax.lax.dot_general(
        xb, wg, (((1,), (1,)), ((), ())), preferred_element_type=jnp.float32)
    u = jax.lax.dot_general(
        xb, wu, (((1,), (1,)), ((), ())), preferred_element_type=jnp.float32)
    a = (g * jax.nn.sigmoid(g) * u).astype(jnp.bfloat16)
    o = jax.lax.dot_general(
        a, wd, (((1,), (1,)), ((), ())), preferred_element_type=jnp.float32)
    comb = comb_ref[...]
    lane = jax.lax.broadcasted_iota(jnp.int32, comb.shape, 1)
    c = jnp.sum(jnp.where(lane == e, comb, 0.0), axis=1, keepdims=True)
    contrib = o * c

    @pl.when(e == 0)
    def _init():
        out_ref[...] = contrib

    @pl.when(e > 0)
    def _acc():
        out_ref[...] += contrib


def kernel(hidden_states, gate_w, Wg, Wu, Wd):
    orig_shape = hidden_states.shape
    x = hidden_states.reshape(-1, H)

    comb = pl.pallas_call(
        _gate_kernel,
        out_shape=jax.ShapeDtypeStruct((T, E), jnp.float32),
        in_specs=[pl.BlockSpec((T, H), lambda: (0, 0)),
                  pl.BlockSpec((E, H), lambda: (0, 0))],
        out_specs=pl.BlockSpec((T, E), lambda: (0, 0)),
    )(x, gate_w)

    xb = x.astype(jnp.bfloat16)
    wgb = Wg.astype(jnp.bfloat16)
    wub = Wu.astype(jnp.bfloat16)
    wdb = Wd.astype(jnp.bfloat16)

    y = pl.pallas_call(
        _moe_kernel,
        grid=(NT, E),
        out_shape=jax.ShapeDtypeStruct((T, H), jnp.float32),
        in_specs=[
            pl.BlockSpec((TT, H), lambda t, e: (t, 0)),
            pl.BlockSpec((TT, E), lambda t, e: (t, 0)),
            pl.BlockSpec((E, F, H), lambda t, e: (0, 0, 0)),
            pl.BlockSpec((E, F, H), lambda t, e: (0, 0, 0)),
            pl.BlockSpec((E, H, F), lambda t, e: (0, 0, 0)),
        ],
        out_specs=pl.BlockSpec((TT, H), lambda t, e: (t, 0)),
        compiler_params=pltpu.CompilerParams(
            dimension_semantics=("arbitrary", "arbitrary")),
    )(xb, comb, wgb, wub, wdb)

    return y.reshape(orig_shape)
